# baseline (speedup 1.0000x reference)
import jax
import jax.numpy as jnp
from jax.experimental import pallas as pl
from jax.experimental.pallas import tpu as pltpu

_N = 10000
_ROWS = 208
_NB = -(-_N // _ROWS)
_CB = 7


def _proj_body(feat_ref, w1_ref, u_ref):
    u = jnp.dot(feat_ref[...], w1_ref[...], preferred_element_type=jnp.float32)
    u_ref[...] = u.astype(jnp.bfloat16)


def _body(u_ref, adj_ref, b1_ref, w2_ref, b2_ref,
          x1_ref, out_ref, v_ref, cache_ref, h2_ref):
    i = pl.program_id(0)

    @pl.when(i < _NB)
    def _():
        a = adj_ref[...].astype(jnp.bfloat16)

        @pl.when(i < _CB)
        def _():
            cache_ref[pl.ds(i * _ROWS, _ROWS), :] = a

        h = jnp.dot(a, u_ref[...], preferred_element_type=jnp.float32)
        x1v = jnp.maximum(h + b1_ref[...], 0.0)
        x1_ref[...] = x1v
        v = jnp.dot(x1v.astype(jnp.bfloat16), w2_ref[...].astype(jnp.bfloat16),
                    preferred_element_type=jnp.float32)
        v_ref[pl.ds(i * _ROWS, _ROWS), :] = v.astype(jnp.bfloat16)

    @pl.when(i >= _NB)
    def _():
        j = i - _NB
        vfull = v_ref[pl.ds(0, _N), :]

        @pl.when(j < _CB)
        def _():
            a = cache_ref[pl.ds(j * _ROWS, _ROWS), :]
            h2_ref[...] = jnp.dot(a, vfull, preferred_element_type=jnp.float32)

        @pl.when(j >= _CB)
        def _():
            a = adj_ref[...].astype(jnp.bfloat16)
            h2_ref[...] = jnp.dot(a, vfull, preferred_element_type=jnp.float32)

        h = h2_ref[...] + b2_ref[...]
        m = jnp.max(h, axis=1, keepdims=True)
        e = jnp.exp(h - m)
        s = jnp.sum(e, axis=1, keepdims=True)
        out_ref[...] = h - m - jnp.log(s)


def kernel(feature, adj, W1, b1, W2, b2):
    f_in = feature.shape[1]
    hid = W1.shape[1]
    dim = W2.shape[1]
    b1r = b1.reshape(1, hid)
    b2r = b2.reshape(1, dim)

    u = pl.pallas_call(
        _proj_body,
        in_specs=[
            pl.BlockSpec((_N, f_in), lambda: (0, 0)),
            pl.BlockSpec((f_in, hid), lambda: (0, 0)),
        ],
        out_specs=pl.BlockSpec((_N, hid), lambda: (0, 0)),
        out_shape=jax.ShapeDtypeStruct((_N, hid), jnp.bfloat16),
    )(feature, W1)

    x1, out = pl.pallas_call(
        _body,
        grid=(2 * _NB,),
        in_specs=[
            pl.BlockSpec((_N, hid), lambda i: (0, 0)),
            pl.BlockSpec((_ROWS, _N),
                         lambda i: (jnp.where(i < _NB, i,
                                              jnp.maximum(i - _NB, _CB)), 0)),
            pl.BlockSpec((1, hid), lambda i: (0, 0)),
            pl.BlockSpec((hid, dim), lambda i: (0, 0)),
            pl.BlockSpec((1, dim), lambda i: (0, 0)),
        ],
        out_specs=[
            pl.BlockSpec((_ROWS, hid),
                         lambda i: (jnp.where(i < _NB, i, _NB - 1), 0)),
            pl.BlockSpec((_ROWS, dim),
                         lambda i: (jnp.where(i < _NB, 0, i - _NB), 0)),
        ],
        out_shape=[
            jax.ShapeDtypeStruct((_N, hid), jnp.float32),
            jax.ShapeDtypeStruct((_N, dim), jnp.float32),
        ],
        scratch_shapes=[
            pltpu.VMEM((_NB * _ROWS, dim), jnp.bfloat16),
            pltpu.VMEM((_CB * _ROWS, _N), jnp.bfloat16),
            pltpu.VMEM((_ROWS, dim), jnp.float32),
        ],
    )(u, adj, b1r, W2, b2r)
    return (x1, out)


# speedup vs baseline: 1.0853x; 1.0853x over previous
import jax
import jax.numpy as jnp
from jax.experimental import pallas as pl
from jax.experimental.pallas import tpu as pltpu

_N = 10000
_ROWS = 200
_ADJ_SPEC = None


def _layer1_body(feat_ref, adj_ref, w1_ref, b1_ref, x1_ref, u_ref):
    @pl.when(pl.program_id(0) == 0)
    def _():
        u = jnp.dot(feat_ref[...], w1_ref[...],
                    preferred_element_type=jnp.float32)
        u_ref[...] = u.astype(jnp.bfloat16)

    a = adj_ref[...].astype(jnp.bfloat16)
    h = jnp.dot(a, u_ref[...], preferred_element_type=jnp.float32)
    x1_ref[...] = jnp.maximum(h + b1_ref[...], 0.0)


def _layer2_body(x1_ref, adj_ref, w2_ref, b2_ref, out_ref, v_ref):
    @pl.when(pl.program_id(0) == 0)
    def _():
        v = jnp.dot(x1_ref[...], w2_ref[...],
                    preferred_element_type=jnp.float32)
        v_ref[...] = v.astype(jnp.bfloat16)

    a = adj_ref[...].astype(jnp.bfloat16)
    h = jnp.dot(a, v_ref[...], preferred_element_type=jnp.float32)
    h = h + b2_ref[...]
    m = jnp.max(h, axis=1, keepdims=True)
    e = jnp.exp(h - m)
    s = jnp.sum(e, axis=1, keepdims=True)
    out_ref[...] = h - m - jnp.log(s)


def kernel(feature, adj, W1, b1, W2, b2):
    f_in = feature.shape[1]
    hid = W1.shape[1]
    dim = W2.shape[1]
    nsteps = _N // _ROWS
    b1r = b1.reshape(1, hid)
    b2r = b2.reshape(1, dim)

    adj_spec = pl.BlockSpec((_ROWS, _N), lambda i: (i, 0))

    x1 = pl.pallas_call(
        _layer1_body,
        grid=(nsteps,),
        in_specs=[
            pl.BlockSpec((_N, f_in), lambda i: (0, 0)),
            adj_spec,
            pl.BlockSpec((f_in, hid), lambda i: (0, 0)),
            pl.BlockSpec((1, hid), lambda i: (0, 0)),
        ],
        out_specs=pl.BlockSpec((_ROWS, hid), lambda i: (i, 0)),
        out_shape=jax.ShapeDtypeStruct((_N, hid), jnp.float32),
        scratch_shapes=[pltpu.VMEM((_N, hid), jnp.bfloat16)],
    )(feature, adj, W1, b1r)

    out = pl.pallas_call(
        _layer2_body,
        grid=(nsteps,),
        in_specs=[
            pl.BlockSpec((_N, hid), lambda i: (0, 0)),
            adj_spec,
            pl.BlockSpec((hid, dim), lambda i: (0, 0)),
            pl.BlockSpec((1, dim), lambda i: (0, 0)),
        ],
        out_specs=pl.BlockSpec((_ROWS, dim), lambda i: (i, 0)),
        out_shape=jax.ShapeDtypeStruct((_N, dim), jnp.float32),
        scratch_shapes=[pltpu.VMEM((_N, dim), jnp.bfloat16)],
    )(x1, adj, W2, b2r)

    return (x1, out)


# speedup vs baseline: 1.1059x; 1.0190x over previous
import jax
import jax.numpy as jnp
from jax.experimental import pallas as pl
from jax.experimental.pallas import tpu as pltpu

_N = 10000
_ROWS = 400
_NB = _N // _ROWS


def _proj_body(feat_ref, w1_ref, u_ref):
    u_ref[...] = jnp.dot(feat_ref[...], w1_ref[...],
                         preferred_element_type=jnp.float32)


def _body(u_ref, adj_ref, b1_ref, w2_ref, b2_ref, x1_ref, out_ref, v_ref):
    i = pl.program_id(0)

    @pl.when(i < _NB)
    def _():
        h = jnp.dot(adj_ref[...], u_ref[...],
                    preferred_element_type=jnp.float32)
        x1v = jnp.maximum(h + b1_ref[...], 0.0)
        x1_ref[...] = x1v
        v_ref[pl.ds(i * _ROWS, _ROWS), :] = jnp.dot(
            x1v, w2_ref[...], preferred_element_type=jnp.float32)

    @pl.when(i >= _NB)
    def _():
        h = jnp.dot(adj_ref[...], v_ref[...],
                    preferred_element_type=jnp.float32)
        h = h + b2_ref[...]
        m = jnp.max(h, axis=1, keepdims=True)
        e = jnp.exp(h - m)
        s = jnp.sum(e, axis=1, keepdims=True)
        out_ref[...] = h - m - jnp.log(s)


def kernel(feature, adj, W1, b1, W2, b2):
    f_in = feature.shape[1]
    hid = W1.shape[1]
    dim = W2.shape[1]
    b1r = b1.reshape(1, hid)
    b2r = b2.reshape(1, dim)

    u = pl.pallas_call(
        _proj_body,
        in_specs=[
            pl.BlockSpec((_N, f_in), lambda: (0, 0)),
            pl.BlockSpec((f_in, hid), lambda: (0, 0)),
        ],
        out_specs=pl.BlockSpec((_N, hid), lambda: (0, 0)),
        out_shape=jax.ShapeDtypeStruct((_N, hid), jnp.float32),
    )(feature, W1)

    x1, out = pl.pallas_call(
        _body,
        grid=(2 * _NB,),
        in_specs=[
            pl.BlockSpec((_N, hid), lambda i: (0, 0)),
            pl.BlockSpec((_ROWS, _N),
                         lambda i: (jnp.where(i < _NB, i, i - _NB), 0)),
            pl.BlockSpec((1, hid), lambda i: (0, 0)),
            pl.BlockSpec((hid, dim), lambda i: (0, 0)),
            pl.BlockSpec((1, dim), lambda i: (0, 0)),
        ],
        out_specs=[
            pl.BlockSpec((_ROWS, hid),
                         lambda i: (jnp.where(i < _NB, i, _NB - 1), 0)),
            pl.BlockSpec((_ROWS, dim),
                         lambda i: (jnp.where(i < _NB, 0, i - _NB), 0)),
        ],
        out_shape=[
            jax.ShapeDtypeStruct((_N, hid), jnp.float32),
            jax.ShapeDtypeStruct((_N, dim), jnp.float32),
        ],
        scratch_shapes=[
            pltpu.VMEM((_N, dim), jnp.float32),
        ],
    )(u, adj, b1r, W2, b2r)
    return (x1, out)
